# baseline (speedup 1.0000x reference)
import jax
import jax.numpy as jnp
from jax import lax
from jax.experimental import pallas as pl
from jax.experimental.pallas import tpu as pltpu
from jax.experimental.pallas import tpu_sc as plsc

_M = 128
_BN = 3584
_NPAD = 100352
_NW = 32
_W = _NPAD // _NW
_PAIR = 2
_L = 16
_MR = _M * _L


def _sc_core(x_hbm, y_hbm, z_hbm, cx_hbm, cy_hbm, cz_hbm,
             hx_hbm, hy_hbm, hz_hbm,
             mind_hbm, nsel_hbm,
             xv, yv, zv, mv, nv, cxv, cyv, czv, n2v, hxv, hyv, hzv):
    wid = lax.axis_index("s") * 2 + lax.axis_index("c")
    base = wid * _W
    pltpu.sync_copy(x_hbm.at[pl.ds(base, _W)], xv)
    pltpu.sync_copy(y_hbm.at[pl.ds(base, _W)], yv)
    pltpu.sync_copy(z_hbm.at[pl.ds(base, _W)], zv)
    pltpu.sync_copy(cx_hbm, cxv)
    pltpu.sync_copy(cy_hbm, cyv)
    pltpu.sync_copy(cz_hbm, czv)
    pltpu.sync_copy(hx_hbm, hxv)
    pltpu.sync_copy(hy_hbm, hyv)
    pltpu.sync_copy(hz_hbm, hzv)

    def norm_step(g, _):
        o = g * _L
        hx = hxv[pl.ds(o, _L)] * jnp.float32(0.5)
        hy = hyv[pl.ds(o, _L)] * jnp.float32(0.5)
        hz = hzv[pl.ds(o, _L)] * jnp.float32(0.5)
        n2v[pl.ds(o, _L)] = (hx * hx + hy * hy) + hz * hz
        return ()

    lax.fori_loop(0, _M, norm_step, (), unroll=False)

    inf16 = jnp.full((_L,), jnp.inf, jnp.float32)

    def step(t, _):
        o = t * (_L * _PAIR)
        xs = [xv[pl.ds(o + _L * p, _L)] for p in range(_PAIR)]
        ys = [yv[pl.ds(o + _L * p, _L)] for p in range(_PAIR)]
        zs = [zv[pl.ds(o + _L * p, _L)] for p in range(_PAIR)]
        ms = [inf16] * _PAIR
        ns = [inf16] * _PAIR
        for j in range(_M):
            cxj = cxv[pl.ds(j * _L, _L)]
            cyj = cyv[pl.ds(j * _L, _L)]
            czj = czv[pl.ds(j * _L, _L)]
            n2j = n2v[pl.ds(j * _L, _L)]
            for p in range(_PAIR):
                dx = xs[p] - cxj
                dy = ys[p] - cyj
                dz = zs[p] - czj
                d2 = (dx * dx + dy * dy) + dz * dz
                lt = d2 < ms[p]
                ms[p] = jnp.where(lt, d2, ms[p])
                ns[p] = jnp.where(lt, n2j, ns[p])
        for p in range(_PAIR):
            mv[pl.ds(o + _L * p, _L)] = ms[p]
            nv[pl.ds(o + _L * p, _L)] = ns[p]
        return ()

    lax.fori_loop(0, _W // (_L * _PAIR), step, (), unroll=False)

    pltpu.sync_copy(mv, mind_hbm.at[pl.ds(base, _W)])
    pltpu.sync_copy(nv, nsel_hbm.at[pl.ds(base, _W)])


def _sc_call(x, y, z, cx, cy, cz, hx, hy, hz):
    mesh = plsc.VectorSubcoreMesh(core_axis_name="c", subcore_axis_name="s")
    f = pl.kernel(
        _sc_core,
        out_type=[
            jax.ShapeDtypeStruct((_NPAD,), jnp.float32),
            jax.ShapeDtypeStruct((_NPAD,), jnp.float32),
        ],
        mesh=mesh,
        scratch_types=[
            pltpu.VMEM((_W,), jnp.float32),
            pltpu.VMEM((_W,), jnp.float32),
            pltpu.VMEM((_W,), jnp.float32),
            pltpu.VMEM((_W,), jnp.float32),
            pltpu.VMEM((_W,), jnp.float32),
            pltpu.VMEM((_MR,), jnp.float32),
            pltpu.VMEM((_MR,), jnp.float32),
            pltpu.VMEM((_MR,), jnp.float32),
            pltpu.VMEM((_MR,), jnp.float32),
            pltpu.VMEM((_MR,), jnp.float32),
            pltpu.VMEM((_MR,), jnp.float32),
            pltpu.VMEM((_MR,), jnp.float32),
        ],
    )
    return f(x, y, z, cx, cy, cz, hx, hy, hz)


def _tc_finish_body(rad_ref, pts_ref, mind2_ref, n2sel_ref,
                    sampled_ref, mind_ref, mask_ref):
    min_dis = jnp.sqrt(mind2_ref[:, :] + jnp.float32(1e-12))
    thresh = jnp.sqrt(n2sel_ref[:, :]) + rad_ref[0]
    mask = min_dis < thresh
    mind_ref[:, :] = min_dis
    mask_ref[:, :] = mask
    sampled_ref[:, :] = jnp.where(mask, pts_ref[:, :], jnp.float32(0.0))


def _tc_finish(rad, pts_t, mind2, n2sel):
    grid = _NPAD // _BN
    return pl.pallas_call(
        _tc_finish_body,
        grid=(grid,),
        in_specs=[
            pl.BlockSpec(memory_space=pltpu.SMEM),
            pl.BlockSpec((3, _BN), lambda i: (0, i)),
            pl.BlockSpec((1, _BN), lambda i: (0, i)),
            pl.BlockSpec((1, _BN), lambda i: (0, i)),
        ],
        out_specs=[
            pl.BlockSpec((3, _BN), lambda i: (0, i)),
            pl.BlockSpec((1, _BN), lambda i: (0, i)),
            pl.BlockSpec((1, _BN), lambda i: (0, i)),
        ],
        out_shape=[
            jax.ShapeDtypeStruct((3, _NPAD), jnp.float32),
            jax.ShapeDtypeStruct((1, _NPAD), jnp.float32),
            jax.ShapeDtypeStruct((1, _NPAD), jnp.bool_),
        ],
    )(rad, pts_t, mind2, n2sel)


@jax.jit
def _run(points, rois, rad):
    n = points.shape[0]
    pts_t = jnp.pad(points.T, ((0, 0), (0, _NPAD - n)))
    rep = jnp.repeat(rois, _L, axis=0)

    mind2, n2sel = _sc_call(
        pts_t[0], pts_t[1], pts_t[2],
        rep[:, 0], rep[:, 1], rep[:, 2],
        rep[:, 3], rep[:, 4], rep[:, 5])

    sampled_t, mind, mask = _tc_finish(
        rad, pts_t, mind2.reshape(1, _NPAD), n2sel.reshape(1, _NPAD))
    return (sampled_t[:, :n].T, mind[0, :n], mask[0, :n])


def kernel(points, rois, sample_radius_with_roi):
    rad = jnp.float32(sample_radius_with_roi).reshape((1,))
    return _run(points, rois, rad)
